# baseline (speedup 1.0000x reference)
import functools

import jax
import jax.numpy as jnp
from jax import lax
from jax.experimental import pallas as pl
from jax.experimental.pallas import tpu as pltpu
from jax.experimental.pallas import tpu_sc as plsc

T = 2048
H = 768
E = 64
I = 768
BLK = 128
L = 12288
NB = L // BLK
NPAIR = 2 * T
Z_COEF = 0.001
LB_COEF = 0.01

NW = 32
TPW = T // NW


def _router_body(x_ref, rw_ref, destA_ref, destB_ref, wA_ref, wB_ref,
                 be_ref, nused_ref, aux_ref):
    x = x_ref[...]
    rw = rw_ref[...]
    logits = jnp.dot(x, rw, preferred_element_type=jnp.float32)
    m = jnp.max(logits, axis=1, keepdims=True)
    ex = jnp.exp(logits - m)
    s = jnp.sum(ex, axis=1, keepdims=True)
    probs = ex / s

    iotaE = lax.broadcasted_iota(jnp.int32, (T, E), 1)
    l1 = jnp.max(logits, axis=1, keepdims=True)
    a1 = jnp.min(jnp.where(logits == l1, iotaE, E), axis=1, keepdims=True)
    logits_m = jnp.where(iotaE == a1, -jnp.inf, logits)
    l2 = jnp.max(logits_m, axis=1, keepdims=True)
    a2 = jnp.min(jnp.where(logits_m == l2, iotaE, E), axis=1, keepdims=True)

    ohA = (iotaE == a1).astype(jnp.float32)
    ohB = (iotaE == a2).astype(jnp.float32)
    p1 = jnp.sum(ohA * probs, axis=1, keepdims=True)
    p2 = jnp.sum(ohB * probs, axis=1, keepdims=True)
    wsum = p1 + p2
    ones16 = jnp.ones((1, 16), jnp.float32)
    wA_ref[...] = (p1 / wsum) * ones16
    wB_ref[...] = (p2 / wsum) * ones16
    counts = jnp.sum(ohA, axis=0, keepdims=True) + jnp.sum(ohB, axis=0,
                                                           keepdims=True)
    pc = jnp.floor((counts + (BLK - 1)) * (1.0 / BLK)).astype(jnp.float32)
    pc = pc * BLK
    c64 = pc
    sh = 1
    while sh < E:
        c64 = c64 + jnp.concatenate(
            [jnp.zeros((1, sh), jnp.float32), c64[:, :E - sh]], axis=1)
        sh *= 2
    off = c64 - pc

    oh = jnp.concatenate([ohA, ohB], axis=0)
    c = oh
    sh = 1
    while sh < NPAIR:
        c = c + jnp.concatenate(
            [jnp.zeros((sh, E), jnp.float32), c[:NPAIR - sh]], axis=0)
        sh *= 2
    cex = c - oh
    dest = jnp.sum((cex + off) * oh, axis=1, keepdims=True)
    desti = dest.astype(jnp.int32)
    destA_ref[...] = desti[:T]
    destB_ref[...] = desti[T:]

    row_start = lax.broadcasted_iota(jnp.int32, (NB, E), 0).astype(
        jnp.float32) * BLK
    ind = jnp.logical_and(row_start >= off, row_start < off + pc)
    lane_e = lax.broadcasted_iota(jnp.int32, (NB, E), 1).astype(jnp.float32)
    be_raw = jnp.sum(jnp.where(ind, lane_e, 0.0), axis=1, keepdims=True)
    used = jnp.sum(ind.astype(jnp.float32), axis=1, keepdims=True) > 0
    e_last = jnp.max(jnp.where(counts > 0,
                               lax.broadcasted_iota(jnp.int32, (1, E), 1)
                               .astype(jnp.float32), -1.0))
    be_ref[...] = jnp.where(used, be_raw, e_last).astype(jnp.int32)
    nused_ref[...] = (jnp.sum(pc, axis=1, keepdims=True)
                      * (1.0 / BLK)).astype(jnp.int32)

    lse = m + jnp.log(s)
    z_loss = Z_COEF * jnp.mean(lse * lse)
    util = counts * (1.0 / T)
    mean_prob = jnp.sum(probs, axis=0, keepdims=True) * (1.0 / T)
    lb_loss = LB_COEF * jnp.sum(util * mean_prob)
    aux_ref[...] = jnp.full((1, 1), 0.0) + z_loss + lb_loss


_router = pl.pallas_call(
    _router_body,
    out_shape=[
        jax.ShapeDtypeStruct((T, 1), jnp.int32),
        jax.ShapeDtypeStruct((T, 1), jnp.int32),
        jax.ShapeDtypeStruct((T, 16), jnp.float32),
        jax.ShapeDtypeStruct((T, 16), jnp.float32),
        jax.ShapeDtypeStruct((NB, 1), jnp.int32),
        jax.ShapeDtypeStruct((1, 1), jnp.int32),
        jax.ShapeDtypeStruct((1, 1), jnp.float32),
    ],
)


def _gmm_body(be_ref, nu_ref, dA_ref, dB_ref, x_ref, g_ref, u_ref, d_ref,
              o_ref):
    i = pl.program_id(0)

    @pl.when(i < nu_ref[0])
    def _():
        slot = lax.broadcasted_iota(jnp.int32, (BLK, T), 0) + i * BLK
        oh = jnp.logical_or(dA_ref[...] == slot,
                            dB_ref[...] == slot).astype(jnp.float32)
        xv = jnp.dot(oh, x_ref[...], preferred_element_type=jnp.float32)
        g = jnp.dot(xv, g_ref[0], preferred_element_type=jnp.float32)
        u = jnp.dot(xv, u_ref[0], preferred_element_type=jnp.float32)
        h = (g / (1.0 + jnp.exp(-g))) * u
        o_ref[...] = jnp.dot(h, d_ref[0], preferred_element_type=jnp.float32)


_gmm = pl.pallas_call(
    _gmm_body,
    grid_spec=pltpu.PrefetchScalarGridSpec(
        num_scalar_prefetch=2,
        grid=(NB,),
        in_specs=[
            pl.BlockSpec((1, T), lambda i, be, nu: (0, 0)),
            pl.BlockSpec((1, T), lambda i, be, nu: (0, 0)),
            pl.BlockSpec((T, H), lambda i, be, nu: (0, 0)),
            pl.BlockSpec((1, H, I), lambda i, be, nu: (be[i], 0, 0)),
            pl.BlockSpec((1, H, I), lambda i, be, nu: (be[i], 0, 0)),
            pl.BlockSpec((1, I, H), lambda i, be, nu: (be[i], 0, 0)),
        ],
        out_specs=pl.BlockSpec(
            (BLK, H), lambda i, be, nu: (jnp.minimum(i, nu[0] - 1), 0)),
    ),
    out_shape=jax.ShapeDtypeStruct((L, H), jnp.float32),
)

@functools.lru_cache(maxsize=1)
def _sc_kernels():
    mesh = plsc.VectorSubcoreMesh(core_axis_name="c", subcore_axis_name="s")

    @functools.partial(
        pl.kernel,
        out_type=jax.ShapeDtypeStruct((T, H), jnp.float32),
        mesh=mesh,
        scratch_types=[
            pltpu.VMEM((TPW, H), jnp.float32),
            pltpu.VMEM((TPW, H), jnp.float32),
            pltpu.VMEM((TPW,), jnp.int32),
            pltpu.VMEM((TPW,), jnp.int32),
            pltpu.VMEM((TPW, 16), jnp.float32),
            pltpu.VMEM((TPW, 16), jnp.float32),
            pltpu.SemaphoreType.DMA,
            pltpu.SemaphoreType.DMA,
        ],
    )
    def combine(y_hbm, destA_hbm, destB_hbm, wA_hbm, wB_hbm, out_hbm,
                bufA, bufB, idxA_v, idxB_v, wa_v, wb_v, semA, semB):
        wid = lax.axis_index("s") * 2 + lax.axis_index("c")
        base = wid * TPW
        pltpu.sync_copy(destA_hbm.at[pl.ds(base, TPW)], idxA_v)
        pltpu.sync_copy(destB_hbm.at[pl.ds(base, TPW)], idxB_v)
        pltpu.sync_copy(wA_hbm.at[pl.ds(base, TPW)], wa_v)
        pltpu.sync_copy(wB_hbm.at[pl.ds(base, TPW)], wb_v)
        cpA = pltpu.async_copy(y_hbm.at[idxA_v], bufA, semA)
        cpB = pltpu.async_copy(y_hbm.at[idxB_v], bufB, semB)
        cpA.wait()
        cpB.wait()

        def row_body(t, carry):
            wa_s = wa_v[t, :]
            wb_s = wb_v[t, :]
            for j in range(H // 16):
                a = bufA[t, pl.ds(j * 16, 16)]
                b = bufB[t, pl.ds(j * 16, 16)]
                bufA[t, pl.ds(j * 16, 16)] = a * wa_s + b * wb_s
            return carry

        lax.fori_loop(0, TPW, row_body, 0)
        pltpu.sync_copy(bufA, out_hbm.at[pl.ds(base, TPW)])

    return combine


def kernel(hidden_states, router_w, gate_w, up_w, down_w):
    b, s, h = hidden_states.shape
    x = hidden_states.reshape(T, H)
    destA2, destB2, wA2, wB2, be2, nused2, aux2 = _router(x, router_w)
    destA = destA2.reshape(T)
    destB = destB2.reshape(T)
    be = be2.reshape(NB)
    nused = nused2.reshape(1)

    combine = _sc_kernels()
    y = _gmm(be, nused, destA2.reshape(1, T), destB2.reshape(1, T), x,
             gate_w, up_w, down_w)
    out = combine(y, destA, destB, wA2, wB2)
    return out.reshape(b, s, h), aux2.reshape(())


# speedup vs baseline: 1.0211x; 1.0211x over previous
import functools

import jax
import jax.numpy as jnp
from jax import lax
from jax.experimental import pallas as pl
from jax.experimental.pallas import tpu as pltpu
from jax.experimental.pallas import tpu_sc as plsc

T = 2048
H = 768
E = 64
I = 768
BLK = 128
L = 12288
NB = L // BLK
NPAIR = 2 * T
Z_COEF = 0.001
LB_COEF = 0.01

NW = 32
TPW = T // NW


def _router_body(x_ref, rw_ref, destA_ref, destB_ref, wA_ref, wB_ref,
                 be_ref, nused_ref, aux_ref):
    x = x_ref[...]
    rw = rw_ref[...]
    logits = jnp.dot(x, rw, preferred_element_type=jnp.float32)
    m = jnp.max(logits, axis=1, keepdims=True)
    ex = jnp.exp(logits - m)
    s = jnp.sum(ex, axis=1, keepdims=True)
    probs = ex / s

    iotaE = lax.broadcasted_iota(jnp.int32, (T, E), 1)
    l1 = jnp.max(logits, axis=1, keepdims=True)
    a1 = jnp.min(jnp.where(logits == l1, iotaE, E), axis=1, keepdims=True)
    logits_m = jnp.where(iotaE == a1, -jnp.inf, logits)
    l2 = jnp.max(logits_m, axis=1, keepdims=True)
    a2 = jnp.min(jnp.where(logits_m == l2, iotaE, E), axis=1, keepdims=True)

    ohA = (iotaE == a1).astype(jnp.float32)
    ohB = (iotaE == a2).astype(jnp.float32)
    p1 = jnp.sum(ohA * probs, axis=1, keepdims=True)
    p2 = jnp.sum(ohB * probs, axis=1, keepdims=True)
    wsum = p1 + p2
    ones16 = jnp.ones((1, 16), jnp.float32)
    wA_ref[...] = (p1 / wsum) * ones16
    wB_ref[...] = (p2 / wsum) * ones16
    counts = jnp.sum(ohA, axis=0, keepdims=True) + jnp.sum(ohB, axis=0,
                                                           keepdims=True)
    pc = jnp.floor((counts + (BLK - 1)) * (1.0 / BLK)).astype(jnp.float32)
    pc = pc * BLK
    c64 = pc
    sh = 1
    while sh < E:
        c64 = c64 + jnp.concatenate(
            [jnp.zeros((1, sh), jnp.float32), c64[:, :E - sh]], axis=1)
        sh *= 2
    off = c64 - pc

    oh = jnp.concatenate([ohA, ohB], axis=0)
    c = oh
    sh = 1
    while sh < NPAIR:
        c = c + jnp.concatenate(
            [jnp.zeros((sh, E), jnp.float32), c[:NPAIR - sh]], axis=0)
        sh *= 2
    cex = c - oh
    dest = jnp.sum((cex + off) * oh, axis=1, keepdims=True)
    desti = dest.astype(jnp.int32)
    destA_ref[...] = desti[:T]
    destB_ref[...] = desti[T:]

    row_start = lax.broadcasted_iota(jnp.int32, (NB, E), 0).astype(
        jnp.float32) * BLK
    ind = jnp.logical_and(row_start >= off, row_start < off + pc)
    lane_e = lax.broadcasted_iota(jnp.int32, (NB, E), 1).astype(jnp.float32)
    be_raw = jnp.sum(jnp.where(ind, lane_e, 0.0), axis=1, keepdims=True)
    used = jnp.sum(ind.astype(jnp.float32), axis=1, keepdims=True) > 0
    e_last = jnp.max(jnp.where(counts > 0,
                               lax.broadcasted_iota(jnp.int32, (1, E), 1)
                               .astype(jnp.float32), -1.0))
    be_ref[...] = jnp.where(used, be_raw, e_last).astype(jnp.int32)
    nused_ref[...] = (jnp.sum(pc, axis=1, keepdims=True)
                      * (1.0 / BLK)).astype(jnp.int32)

    lse = m + jnp.log(s)
    z_loss = Z_COEF * jnp.mean(lse * lse)
    util = counts * (1.0 / T)
    mean_prob = jnp.sum(probs, axis=0, keepdims=True) * (1.0 / T)
    lb_loss = LB_COEF * jnp.sum(util * mean_prob)
    aux_ref[...] = jnp.full((1, 1), 0.0) + z_loss + lb_loss


_router = pl.pallas_call(
    _router_body,
    out_shape=[
        jax.ShapeDtypeStruct((T, 1), jnp.int32),
        jax.ShapeDtypeStruct((T, 1), jnp.int32),
        jax.ShapeDtypeStruct((T, 16), jnp.float32),
        jax.ShapeDtypeStruct((T, 16), jnp.float32),
        jax.ShapeDtypeStruct((NB, 1), jnp.int32),
        jax.ShapeDtypeStruct((1, 1), jnp.int32),
        jax.ShapeDtypeStruct((1, 1), jnp.float32),
    ],
)


def _gmm_body(be_ref, nu_ref, x_ref, g_ref, u_ref, d_ref, o_ref):
    i = pl.program_id(0)

    @pl.when(i < nu_ref[0])
    def _():
        xv = x_ref[...]
        g = jnp.dot(xv, g_ref[0], preferred_element_type=jnp.float32)
        u = jnp.dot(xv, u_ref[0], preferred_element_type=jnp.float32)
        h = (g / (1.0 + jnp.exp(-g))) * u
        o_ref[...] = jnp.dot(h, d_ref[0], preferred_element_type=jnp.float32)


_gmm = pl.pallas_call(
    _gmm_body,
    grid_spec=pltpu.PrefetchScalarGridSpec(
        num_scalar_prefetch=2,
        grid=(NB,),
        in_specs=[
            pl.BlockSpec((BLK, H),
                         lambda i, be, nu: (jnp.minimum(i, nu[0] - 1), 0)),
            pl.BlockSpec((1, H, I), lambda i, be, nu: (be[i], 0, 0)),
            pl.BlockSpec((1, H, I), lambda i, be, nu: (be[i], 0, 0)),
            pl.BlockSpec((1, I, H), lambda i, be, nu: (be[i], 0, 0)),
        ],
        out_specs=pl.BlockSpec(
            (BLK, H), lambda i, be, nu: (jnp.minimum(i, nu[0] - 1), 0)),
    ),
    out_shape=jax.ShapeDtypeStruct((L, H), jnp.float32),
)

@functools.lru_cache(maxsize=1)
def _sc_kernels():
    mesh = plsc.VectorSubcoreMesh(core_axis_name="c", subcore_axis_name="s")

    @functools.partial(
        pl.kernel,
        out_type=jax.ShapeDtypeStruct((L, H), jnp.float32),
        mesh=mesh,
        scratch_types=[
            pltpu.VMEM((TPW, H), jnp.float32),
            pltpu.VMEM((TPW,), jnp.int32),
            pltpu.VMEM((TPW,), jnp.int32),
            pltpu.SemaphoreType.DMA,
        ],
    )
    def dispatch(x_hbm, destA_hbm, destB_hbm, out_hbm, rows_v, idxA_v,
                 idxB_v, sem):
        wid = lax.axis_index("s") * 2 + lax.axis_index("c")
        base = wid * TPW
        pltpu.sync_copy(x_hbm.at[pl.ds(base, TPW)], rows_v)
        pltpu.sync_copy(destA_hbm.at[pl.ds(base, TPW)], idxA_v)
        pltpu.sync_copy(destB_hbm.at[pl.ds(base, TPW)], idxB_v)
        pltpu.async_copy(rows_v, out_hbm.at[idxA_v], sem).wait()
        pltpu.async_copy(rows_v, out_hbm.at[idxB_v], sem).wait()

    HLF = TPW // 2

    @functools.partial(
        pl.kernel,
        out_type=jax.ShapeDtypeStruct((T, H), jnp.float32),
        mesh=mesh,
        scratch_types=[
            pltpu.VMEM((TPW, H), jnp.float32),
            pltpu.VMEM((TPW, H), jnp.float32),
            pltpu.VMEM((TPW,), jnp.int32),
            pltpu.VMEM((TPW,), jnp.int32),
            pltpu.VMEM((TPW, 16), jnp.float32),
            pltpu.VMEM((TPW, 16), jnp.float32),
            pltpu.SemaphoreType.DMA,
            pltpu.SemaphoreType.DMA,
        ],
    )
    def combine(y_hbm, destA_hbm, destB_hbm, wA_hbm, wB_hbm, out_hbm,
                bufA, bufB, idxA_v, idxB_v, wa_v, wb_v, semA, semB):
        wid = lax.axis_index("s") * 2 + lax.axis_index("c")
        base = wid * TPW
        pltpu.sync_copy(destA_hbm.at[pl.ds(base, TPW)], idxA_v)
        pltpu.sync_copy(destB_hbm.at[pl.ds(base, TPW)], idxB_v)
        pltpu.sync_copy(wA_hbm.at[pl.ds(base, TPW)], wa_v)
        pltpu.sync_copy(wB_hbm.at[pl.ds(base, TPW)], wb_v)
        cps = []
        for h0, sem in ((0, semA), (HLF, semB)):
            cps.append(pltpu.async_copy(
                y_hbm.at[idxA_v.at[pl.ds(h0, HLF)]],
                bufA.at[pl.ds(h0, HLF)], sem))
            cps.append(pltpu.async_copy(
                y_hbm.at[idxB_v.at[pl.ds(h0, HLF)]],
                bufB.at[pl.ds(h0, HLF)], sem))

        def row_body(t, carry):
            wa_s = wa_v[t, :]
            wb_s = wb_v[t, :]
            for j in range(H // 16):
                a = bufA[t, pl.ds(j * 16, 16)]
                b = bufB[t, pl.ds(j * 16, 16)]
                bufA[t, pl.ds(j * 16, 16)] = a * wa_s + b * wb_s
            return carry

        cps[0].wait()
        cps[1].wait()
        lax.fori_loop(0, HLF, row_body, 0)
        wb_out = pltpu.async_copy(bufA.at[pl.ds(0, HLF)],
                                  out_hbm.at[pl.ds(base, HLF)], semA)
        cps[2].wait()
        cps[3].wait()
        lax.fori_loop(HLF, TPW, row_body, 0)
        wb_out.wait()
        pltpu.sync_copy(bufA.at[pl.ds(HLF, HLF)],
                        out_hbm.at[pl.ds(base + HLF, HLF)])

    return dispatch, combine


def kernel(hidden_states, router_w, gate_w, up_w, down_w):
    b, s, h = hidden_states.shape
    x = hidden_states.reshape(T, H)
    destA2, destB2, wA2, wB2, be2, nused2, aux2 = _router(x, router_w)
    destA = destA2.reshape(T)
    destB = destB2.reshape(T)
    be = be2.reshape(NB)
    nused = nused2.reshape(1)

    dispatch, combine = _sc_kernels()
    sorted_x = dispatch(x, destA, destB)
    y = _gmm(be, nused, sorted_x, gate_w, up_w, down_w)
    out = combine(y, destA, destB, wA2, wB2)
    return out.reshape(b, s, h), aux2.reshape(())
